# Initial kernel scaffold; baseline (speedup 1.0000x reference)
import jax
import jax.numpy as jnp
from jax.experimental import pallas as pl


def kernel(X, Y, table):
    raise NotImplementedError("write your pallas kernel here")


# speedup vs baseline: 1.4850x; 1.4850x over previous
import jax
import jax.numpy as jnp
from jax import lax
from jax.experimental import pallas as pl
from jax.experimental.pallas import tpu as pltpu

VOCAB = 1000
B, L = 1024, 20
TOK = B * L
BLK = 256
NBLK = TOK // BLK


def _kernel_body(x_ref, y_ref, table_ref, out_ref, loss_ref, lse_scr, acc_scr):
    pid = pl.program_id(0)
    t = table_ref[...]

    @pl.when(pid == 0)
    def _init():
        m = jnp.max(t, axis=1, keepdims=True)
        s = jnp.sum(jnp.exp(t - m), axis=1, keepdims=True)
        lse_scr[...] = m + jnp.log(s)
        acc_scr[0] = 0.0

    xv = x_ref[0]
    yv = y_ref[0]
    iota = lax.broadcasted_iota(jnp.int32, (BLK, VOCAB), 1)
    oh_x = (xv == iota).astype(jnp.float32)
    oh_y = (yv == iota).astype(jnp.float32)

    logits = lax.dot_general(
        oh_x, t, (((1,), (0,)), ((), ())),
        preferred_element_type=jnp.float32)
    out_ref[0] = logits

    lse_tok = lax.dot_general(
        oh_x, lse_scr[...], (((1,), (0,)), ((), ())),
        preferred_element_type=jnp.float32)
    picked = jnp.sum(logits * oh_y, axis=1, keepdims=True)
    acc_scr[0] += jnp.sum(lse_tok - picked)

    @pl.when(pid == NBLK - 1)
    def _fin():
        loss_ref[...] = jnp.full((1, 1), acc_scr[0] / TOK, jnp.float32)


def kernel(X, Y, table):
    Xr = X.astype(jnp.int32).reshape(NBLK, BLK, 1)
    Yr = Y.astype(jnp.int32).reshape(NBLK, BLK, 1)
    logits3, loss = pl.pallas_call(
        _kernel_body,
        grid=(NBLK,),
        in_specs=[
            pl.BlockSpec((1, BLK, 1), lambda i: (i, 0, 0)),
            pl.BlockSpec((1, BLK, 1), lambda i: (i, 0, 0)),
            pl.BlockSpec((VOCAB, VOCAB), lambda i: (0, 0)),
        ],
        out_specs=[
            pl.BlockSpec((1, BLK, VOCAB), lambda i: (i, 0, 0)),
            pl.BlockSpec((1, 1), lambda i: (0, 0)),
        ],
        out_shape=[
            jax.ShapeDtypeStruct((NBLK, BLK, VOCAB), jnp.float32),
            jax.ShapeDtypeStruct((1, 1), jnp.float32),
        ],
        scratch_shapes=[
            pltpu.VMEM((VOCAB, 1), jnp.float32),
            pltpu.SMEM((1,), jnp.float32),
        ],
    )(Xr, Yr, table)
    return logits3.reshape(B, L, VOCAB), loss[0, 0]


# speedup vs baseline: 1.5273x; 1.0285x over previous
import functools

import jax
import jax.numpy as jnp
from jax import lax
from jax.experimental import pallas as pl
from jax.experimental.pallas import tpu as pltpu
from jax.experimental.pallas import tpu_sc as plsc

VOCAB = 1000
B, L = 1024, 20
TOK = B * L

NC, NS = 2, 16
NW = NC * NS
BPW = TOK // NW
CHUNK = 64
NCH = BPW // CHUNK
LANES = 16

BLK = 256
NBLK = TOK // BLK


def _sc_gather_body(table_hbm, x_hbm, y_hbm, out_hbm, part_hbm,
                    x_v, y_v, rows0, rows1, pick_v, sem0, sem1):
    wid = lax.axis_index("s") * NC + lax.axis_index("c")
    base = pl.multiple_of(wid * BPW, BPW)

    pltpu.sync_copy(x_hbm.at[pl.ds(base, BPW)], x_v)
    pltpu.sync_copy(y_hbm.at[pl.ds(base, BPW)], y_v)

    rows = (rows0, rows1)
    sems = (sem0, sem1)
    cps = [None, None]
    acc = jnp.zeros((LANES,), jnp.float32)

    for k in range(NCH + 1):
        if k < NCH:
            sl = k % 2
            cps[sl] = pltpu.async_copy(
                table_hbm.at[x_v.at[pl.ds(k * CHUNK, CHUNK)]],
                rows[sl], sems[sl])
        if k >= 1:
            j = k - 1
            sl = j % 2
            cps[sl].wait()
            for q in range(CHUNK // LANES):
                rid = lax.iota(jnp.int32, 16) + jnp.int32(q * LANES)
                y16 = y_v[pl.ds(j * CHUNK + q * LANES, LANES)]
                acc = acc + plsc.load_gather(rows[sl], [rid, y16])
            pltpu.sync_copy(rows[sl], out_hbm.at[pl.ds(base + j * CHUNK, CHUNK)])

    pick_v[...] = acc
    pltpu.sync_copy(pick_v, part_hbm.at[wid])


_sc_gather = functools.partial(
    pl.kernel,
    out_type=[
        jax.ShapeDtypeStruct((TOK, VOCAB), jnp.float32),
        jax.ShapeDtypeStruct((NW, LANES), jnp.float32),
    ],
    mesh=plsc.VectorSubcoreMesh(
        core_axis_name="c", subcore_axis_name="s",
        num_cores=NC, num_subcores=NS),
    compiler_params=pltpu.CompilerParams(
        use_tc_tiling_on_sc=False, needs_layout_passes=False),
    scratch_types=[
        pltpu.VMEM((BPW,), jnp.int32),
        pltpu.VMEM((BPW,), jnp.int32),
        pltpu.VMEM((CHUNK, VOCAB), jnp.float32),
        pltpu.VMEM((CHUNK, VOCAB), jnp.float32),
        pltpu.VMEM((LANES,), jnp.float32),
        pltpu.SemaphoreType.DMA,
        pltpu.SemaphoreType.DMA,
    ],
)(_sc_gather_body)


def _lse_body(x_ref, table_ref, out_ref, lse_scr, acc_scr):
    pid = pl.program_id(0)
    t = table_ref[...]

    @pl.when(pid == 0)
    def _init():
        m = jnp.max(t, axis=1, keepdims=True)
        s = jnp.sum(jnp.exp(t - m), axis=1, keepdims=True)
        lse_scr[...] = m + jnp.log(s)
        acc_scr[0] = 0.0

    xv = x_ref[0]
    iota = lax.broadcasted_iota(jnp.int32, (BLK, VOCAB), 1)
    oh_x = (xv == iota).astype(jnp.float32)
    lse_tok = lax.dot_general(
        oh_x, lse_scr[...], (((1,), (0,)), ((), ())),
        preferred_element_type=jnp.float32)
    acc_scr[0] += jnp.sum(lse_tok)

    @pl.when(pid == NBLK - 1)
    def _fin():
        out_ref[...] = jnp.full((1, 1), acc_scr[0], jnp.float32)


def _lse_sum(Xr, table):
    return pl.pallas_call(
        _lse_body,
        grid=(NBLK,),
        in_specs=[
            pl.BlockSpec((1, BLK, 1), lambda i: (i, 0, 0)),
            pl.BlockSpec((VOCAB, VOCAB), lambda i: (0, 0)),
        ],
        out_specs=pl.BlockSpec((1, 1), lambda i: (0, 0)),
        out_shape=jax.ShapeDtypeStruct((1, 1), jnp.float32),
        scratch_shapes=[
            pltpu.VMEM((VOCAB, 1), jnp.float32),
            pltpu.SMEM((1,), jnp.float32),
        ],
    )(Xr, table)


def kernel(X, Y, table):
    Xf = X.astype(jnp.int32).reshape(TOK)
    Yf = Y.astype(jnp.int32).reshape(TOK)
    logits_flat, parts = _sc_gather(table, Xf, Yf)
    lse_sum = _lse_sum(X.astype(jnp.int32).reshape(NBLK, BLK, 1), table)
    loss = (lse_sum[0, 0] - jnp.sum(parts)) / TOK
    return logits_flat.reshape(B, L, VOCAB), loss
